# baseline (speedup 1.0000x reference)
import functools

import jax
import jax.numpy as jnp
from jax import lax
from jax.experimental import pallas as pl
from jax.experimental.pallas import tpu as pltpu
from jax.experimental.pallas import tpu_sc as plsc

B = 16384
D = 64
NC = 2
NS = 16
NW = NC * NS
BPW = B // NW
CH = 128
NCHUNK = BPW // CH
LANES = 16

_SIN_COEFFS = (
    -1.0 / 1307674368000.0,
    1.0 / 6227020800.0,
    -1.0 / 39916800.0,
    1.0 / 362880.0,
    -1.0 / 5040.0,
    1.0 / 120.0,
    -1.0 / 6.0,
    1.0,
)
_COS_COEFFS = (
    -1.0 / 87178291200.0,
    1.0 / 479001600.0,
    -1.0 / 3628800.0,
    1.0 / 40320.0,
    -1.0 / 720.0,
    1.0 / 24.0,
    -0.5,
    1.0,
)


def _sincos(x):
    z = x * x
    s = jnp.float32(_SIN_COEFFS[0])
    for c in _SIN_COEFFS[1:]:
        s = s * z + jnp.float32(c)
    s = s * x
    c_acc = jnp.float32(_COS_COEFFS[0])
    for c in _COS_COEFFS[1:]:
        c_acc = c_acc * z + jnp.float32(c)
    return s, c_acc


_mesh = plsc.VectorSubcoreMesh(core_axis_name="c", subcore_axis_name="s")


@functools.partial(
    pl.kernel,
    mesh=_mesh,
    compiler_params=pltpu.CompilerParams(needs_layout_passes=False),
    out_type=(
        jax.ShapeDtypeStruct((D, B), jnp.float32),
        jax.ShapeDtypeStruct((D, B), jnp.float32),
    ),
    scratch_types=[
        pltpu.VMEM((BPW,), jnp.int32),
        pltpu.VMEM((BPW,), jnp.int32),
        pltpu.VMEM((BPW,), jnp.int32),
        pltpu.VMEM((BPW,), jnp.int32),
        pltpu.VMEM((CH, 2 * D), jnp.float32),
        pltpu.VMEM((CH, 2 * D), jnp.float32),
        pltpu.VMEM((CH, 2 * D), jnp.float32),
        pltpu.VMEM((D, BPW), jnp.float32),
        pltpu.VMEM((D, BPW), jnp.float32),
        pltpu.SemaphoreType.DMA,
    ],
)
def _rotate_sc(e1_hbm, r_hbm, er2_hbm, ei2_hbm, rel2_hbm, outr_hbm, outi_hbm,
               idx1_v, idx2_v, p1_v, p2_v, erp_v, eip_v, thp_v, or_v, oi_v,
               sem):
    wid = lax.axis_index("s") * NC + lax.axis_index("c")
    base = wid * BPW

    pltpu.sync_copy(e1_hbm.at[pl.ds(base, BPW)], idx1_v)
    pltpu.sync_copy(r_hbm.at[pl.ds(base, BPW)], idx2_v)

    for v in range(BPW // LANES):
        sl = pl.ds(v * LANES, LANES)
        p1_v[sl] = lax.shift_right_logical(idx1_v[sl], 1)
        p2_v[sl] = lax.shift_right_logical(idx2_v[sl], 1)

    iota = lax.iota(jnp.int32, LANES)

    for chunk in range(NCHUNK):
        csl = pl.ds(chunk * CH, CH)
        cps = [
            pltpu.async_copy(er2_hbm.at[p1_v.at[csl]], erp_v, sem),
            pltpu.async_copy(ei2_hbm.at[p1_v.at[csl]], eip_v, sem),
            pltpu.async_copy(rel2_hbm.at[p2_v.at[csl]], thp_v, sem),
        ]
        for cp in cps:
            cp.wait()

        for bv in range(CH // LANES):
            gcol = chunk * CH + bv * LANES
            gsl = pl.ds(gcol, LANES)
            row_i = iota + bv * LANES
            par1 = lax.shift_left(idx1_v[gsl] & 1, 6)
            par2 = lax.shift_left(idx2_v[gsl] & 1, 6)

            def fbody(f, _, row_i=row_i, par1=par1, par2=par2, gsl=gsl):
                c1 = par1 + f
                c2 = par2 + f
                a = plsc.load_gather(erp_v, [row_i, c1])
                b = plsc.load_gather(eip_v, [row_i, c1])
                theta = plsc.load_gather(thp_v, [row_i, c2])
                s, c = _sincos(theta)
                or_v[f, gsl] = a * c - b * s
                oi_v[f, gsl] = a * s + b * c
                return _

            lax.fori_loop(0, D, fbody, None)

    pltpu.sync_copy(or_v, outr_hbm.at[:, pl.ds(base, BPW)])
    pltpu.sync_copy(oi_v, outi_hbm.at[:, pl.ds(base, BPW)])


def kernel(e1, r, entity_real, entity_img, relation):
    e1 = e1.astype(jnp.int32)
    r = r.astype(jnp.int32)
    er2 = entity_real.reshape(500000, 128)
    ei2 = entity_img.reshape(500000, 128)
    rel2 = relation.reshape(500, 128)
    outr_t, outi_t = _rotate_sc(e1, r, er2, ei2, rel2)
    return outr_t.T, outi_t.T
